# baseline (speedup 1.0000x reference)
import jax
import jax.numpy as jnp
from jax import lax
from jax.experimental import pallas as pl
from jax.experimental.pallas import tpu as pltpu
from jax.experimental.pallas import tpu_sc as plsc

NC, NS, L = 2, 16, 16
NW = NC * NS
B, S = 4096, 50
D = 64
TOT = B * S
CHUNK = 128
ROWS = TOT // CHUNK
RPW = ROWS // NW


def _emb_body(idx_hbm, table_hbm, emb_hbm, mask_hbm, idx_v, mask_v, buf0, buf1, sem0, sem1):
    wid = lax.axis_index("s") * NC + lax.axis_index("c")
    base = wid * RPW

    pltpu.sync_copy(idx_hbm.at[wid], idx_v)

    def mask_iter(i, carry):
        r = i // (CHUNK // L)
        c = (i % (CHUNK // L)) * L
        v = idx_v[r, pl.ds(c, L)]
        mask_v[r, pl.ds(c, L)] = jnp.where(v > 0, 1.0, 0.0).astype(jnp.float32)
        return carry
    lax.fori_loop(0, RPW * (CHUNK // L), mask_iter, 0)
    pltpu.sync_copy(mask_v, mask_hbm.at[wid])

    bufs = (buf0, buf1)
    sems = (sem0, sem1)

    gather0 = pltpu.async_copy(table_hbm.at[idx_v.at[0]], buf0, sem0)
    gather0.wait()

    def chunk_iter(j, carry):
        for p in range(2):
            @pl.when(j % 2 == p)
            def _do():
                g = pltpu.async_copy(table_hbm.at[idx_v.at[j]], bufs[p], sems[p])
                pltpu.sync_copy(bufs[1 - p], emb_hbm.at[pl.ds((base + j - 1) * CHUNK, CHUNK)])
                g.wait()
        return carry
    lax.fori_loop(1, RPW, chunk_iter, 0)

    pltpu.sync_copy(bufs[(RPW - 1) % 2], emb_hbm.at[pl.ds((base + RPW - 1) * CHUNK, CHUNK)])


def kernel(string_lkup, table):
    idx_flat = string_lkup.reshape(NW, RPW, CHUNK)
    mesh = plsc.VectorSubcoreMesh(core_axis_name="c", subcore_axis_name="s")
    emb_flat, mask_flat = pl.kernel(
        _emb_body,
        out_type=[
            jax.ShapeDtypeStruct((TOT, D), jnp.float32),
            jax.ShapeDtypeStruct((NW, RPW, CHUNK), jnp.float32),
        ],
        mesh=mesh,
        compiler_params=pltpu.CompilerParams(use_tc_tiling_on_sc=False),
        scratch_types=[
            pltpu.VMEM((RPW, CHUNK), jnp.int32),
            pltpu.VMEM((RPW, CHUNK), jnp.float32),
            pltpu.VMEM((CHUNK, D), jnp.float32),
            pltpu.VMEM((CHUNK, D), jnp.float32),
            pltpu.SemaphoreType.DMA,
            pltpu.SemaphoreType.DMA,
        ],
    )(idx_flat, table)
    return emb_flat.reshape(B, S, D), mask_flat.reshape(B, S)
